# speedup vs baseline: 4.5684x; 4.5684x over previous
import jax
import jax.numpy as jnp
from jax.experimental import pallas as pl

_LANE = 128


def _cumsum_kernel(x_ref, o_ref):
    n = x_ref.shape[1]
    chunks = n // _LANE

    ri = jax.lax.broadcasted_iota(jnp.int32, (_LANE, _LANE), 0)
    ci = jax.lax.broadcasted_iota(jnp.int32, (_LANE, _LANE), 1)
    tri = (ri <= ci).astype(jnp.float32)

    carry = jnp.zeros((x_ref.shape[0], 1), jnp.float32)
    for c in range(chunks):
        xc = x_ref[:, c * _LANE:(c + 1) * _LANE]
        yc = jax.lax.dot_general(
            xc, tri, (((1,), (0,)), ((), ())),
            precision=jax.lax.Precision.HIGHEST,
            preferred_element_type=jnp.float32)
        o_ref[:, c * _LANE:(c + 1) * _LANE] = yc + carry
        carry = carry + yc[:, _LANE - 1:_LANE]


def _cumsum_rows(x2d, block_rows, interpret=False):
    rows, n = x2d.shape
    grid = (rows // block_rows,)
    return pl.pallas_call(
        _cumsum_kernel,
        grid=grid,
        in_specs=[pl.BlockSpec((block_rows, n), lambda i: (i, 0))],
        out_specs=pl.BlockSpec((block_rows, n), lambda i: (i, 0)),
        out_shape=jax.ShapeDtypeStruct((rows, n), jnp.float32),
        interpret=interpret,
    )(x2d)


def kernel(x):
    b, s, n = x.shape
    x2d = x.reshape(b * s, n).astype(jnp.float32)
    out = _cumsum_rows(x2d, block_rows=256)
    return out.reshape(b, s, n).astype(x.dtype)


# speedup vs baseline: 6.2706x; 1.3726x over previous
import jax
import jax.numpy as jnp
from jax.experimental import pallas as pl

_LANE = 128


def _cumsum_kernel(x_ref, o_ref):
    n = x_ref.shape[1]
    chunks = n // _LANE

    ri = jax.lax.broadcasted_iota(jnp.int32, (_LANE, _LANE), 0)
    ci = jax.lax.broadcasted_iota(jnp.int32, (_LANE, _LANE), 1)
    tri = (ri <= ci).astype(jnp.bfloat16)

    dims = (((1,), (0,)), ((), ()))
    carry = jnp.zeros((x_ref.shape[0], 1), jnp.float32)
    for c in range(chunks):
        xc = x_ref[:, c * _LANE:(c + 1) * _LANE]
        hi = xc.astype(jnp.bfloat16)
        lo = (xc - hi.astype(jnp.float32)).astype(jnp.bfloat16)
        yc = jax.lax.dot_general(
            hi, tri, dims, preferred_element_type=jnp.float32)
        yc = yc + jax.lax.dot_general(
            lo, tri, dims, preferred_element_type=jnp.float32)
        o_ref[:, c * _LANE:(c + 1) * _LANE] = yc + carry
        carry = carry + yc[:, _LANE - 1:_LANE]


def _cumsum_rows(x2d, block_rows, interpret=False):
    rows, n = x2d.shape
    grid = (rows // block_rows,)
    return pl.pallas_call(
        _cumsum_kernel,
        grid=grid,
        in_specs=[pl.BlockSpec((block_rows, n), lambda i: (i, 0))],
        out_specs=pl.BlockSpec((block_rows, n), lambda i: (i, 0)),
        out_shape=jax.ShapeDtypeStruct((rows, n), jnp.float32),
        interpret=interpret,
    )(x2d)


def kernel(x):
    b, s, n = x.shape
    x2d = x.reshape(b * s, n).astype(jnp.float32)
    out = _cumsum_rows(x2d, block_rows=256)
    return out.reshape(b, s, n).astype(x.dtype)


# speedup vs baseline: 6.5278x; 1.0410x over previous
import jax
import jax.numpy as jnp
from jax.experimental import pallas as pl

_LANE = 128


def _cumsum_kernel(x_ref, o_ref):
    n = x_ref.shape[1]
    chunks = n // _LANE

    ri = jax.lax.broadcasted_iota(jnp.int32, (_LANE, _LANE), 0)
    ci = jax.lax.broadcasted_iota(jnp.int32, (_LANE, _LANE), 1)
    tri = (ri <= ci).astype(jnp.bfloat16)

    dims = (((1,), (0,)), ((), ()))
    carry = jnp.zeros((x_ref.shape[0], 1), jnp.float32)
    for c in range(chunks):
        xc = x_ref[:, c * _LANE:(c + 1) * _LANE]
        hi = xc.astype(jnp.bfloat16)
        lo = (xc - hi.astype(jnp.float32)).astype(jnp.bfloat16)
        yc = jax.lax.dot_general(
            hi, tri, dims, preferred_element_type=jnp.float32)
        yc = yc + jax.lax.dot_general(
            lo, tri, dims, preferred_element_type=jnp.float32)
        o_ref[:, c * _LANE:(c + 1) * _LANE] = yc + carry
        carry = carry + yc[:, _LANE - 1:_LANE]


def _cumsum_rows(x2d, block_rows, interpret=False):
    rows, n = x2d.shape
    grid = (rows // block_rows,)
    return pl.pallas_call(
        _cumsum_kernel,
        grid=grid,
        in_specs=[pl.BlockSpec((block_rows, n), lambda i: (i, 0))],
        out_specs=pl.BlockSpec((block_rows, n), lambda i: (i, 0)),
        out_shape=jax.ShapeDtypeStruct((rows, n), jnp.float32),
        interpret=interpret,
    )(x2d)


def kernel(x):
    b, s, n = x.shape
    x2d = x.reshape(b * s, n).astype(jnp.float32)
    out = _cumsum_rows(x2d, block_rows=512)
    return out.reshape(b, s, n).astype(x.dtype)
